# Initial kernel scaffold; baseline (speedup 1.0000x reference)
import jax
import jax.numpy as jnp
from jax.experimental import pallas as pl


def kernel(inputs, img_feat0, img_feat1, img_feat2, img_feat3):
    raise NotImplementedError("write your pallas kernel here")


# speedup vs baseline: 3.2669x; 3.2669x over previous
import functools

import jax
import jax.numpy as jnp
from jax import lax
from jax.experimental import pallas as pl
from jax.experimental.pallas import tpu as pltpu
from jax.experimental.pallas import tpu_sc as plsc

_B = 16
_CH = (64, 128, 256, 512)
_SS = (56, 28, 14, 7)
_COL = (3, 67, 195, 451)
_OUT_D = 963


@functools.lru_cache(maxsize=None)
def _make_sc_kernel(n):
    assert n % _B == 0
    info = plsc.get_sparse_core_info()
    nc, ns = info.num_cores, info.num_subcores
    nw = nc * ns
    nblk = n // _B
    trips = (nblk + nw - 1) // nw

    mesh = plsc.VectorSubcoreMesh(core_axis_name="c", subcore_axis_name="s")

    scratch = [
        pltpu.VMEM((_B * 3,), jnp.float32),
        pltpu.VMEM((16, _B), jnp.int32),
        pltpu.VMEM((16 * _B,), jnp.float32),
        pltpu.VMEM((_B * _OUT_D,), jnp.float32),
    ]
    for c in _CH:
        for _ in range(4):
            scratch.append(pltpu.VMEM((_B, c), jnp.float32))
    scratch.append(pltpu.SemaphoreType.DMA)

    @functools.partial(
        pl.kernel,
        mesh=mesh,
        out_type=jax.ShapeDtypeStruct((n * _OUT_D,), jnp.float32),
        scratch_types=scratch,
        compiler_params=pltpu.CompilerParams(
            needs_layout_passes=False, use_tc_tiling_on_sc=False),
    )
    def sc_kernel(coord_hbm, t0, t1, t2, t3, out_hbm,
                  coord_v, idx2d, w_v, out_v, *rest):
        bufs = rest[:-1]
        sem = rest[-1]
        tables = (t0, t1, t2, t3)
        wid = lax.axis_index("s") * nc + lax.axis_index("c")
        lane = lax.iota(jnp.int32, 16)

        def trip_body(t, carry):
            bid = t * nw + wid

            @pl.when(bid < nblk)
            def _():
                vbase = bid * _B
                pltpu.sync_copy(coord_hbm.at[pl.ds(vbase * 3, _B * 3)],
                                coord_v)
                lane3 = lane * 3
                xc = plsc.load_gather(coord_v, [lane3])
                yc = plsc.load_gather(coord_v, [lane3 + 1])
                zc = plsc.load_gather(coord_v, [lane3 + 2])
                nz = -zc
                h = 250.0 * (-yc / nz) + 112.0
                w = 250.0 * (xc / nz) + 112.0
                h = jnp.minimum(jnp.maximum(h, 0.0), 223.0)
                w = jnp.minimum(jnp.maximum(w, 0.0), 223.0)
                for m in range(4):
                    s = _SS[m]
                    x = h * (s / 224.0)
                    y = w * (s / 224.0)
                    xi1 = x.astype(jnp.int32)
                    x1f = xi1.astype(jnp.float32)
                    xi2 = jnp.minimum(xi1 + (x > x1f).astype(jnp.int32), s - 1)
                    x2f = xi2.astype(jnp.float32)
                    yi1 = y.astype(jnp.int32)
                    y1f = yi1.astype(jnp.float32)
                    yi2 = jnp.minimum(yi1 + (y > y1f).astype(jnp.int32), s - 1)
                    y2f = yi2.astype(jnp.float32)
                    idx2d[4 * m + 0, :] = xi1 * s + yi1
                    idx2d[4 * m + 1, :] = xi2 * s + yi1
                    idx2d[4 * m + 2, :] = xi1 * s + yi2
                    idx2d[4 * m + 3, :] = xi2 * s + yi2
                    w_v[pl.ds((4 * m + 0) * 16, 16)] = (x2f - x) * (y2f - y)
                    w_v[pl.ds((4 * m + 1) * 16, 16)] = (x - x1f) * (y2f - y)
                    w_v[pl.ds((4 * m + 2) * 16, 16)] = (x2f - x) * (y - y1f)
                    w_v[pl.ds((4 * m + 3) * 16, 16)] = (x - x1f) * (y - y1f)
                copies = []
                for m in range(4):
                    for c in range(4):
                        copies.append(pltpu.async_copy(
                            tables[m].at[idx2d.at[4 * m + c]],
                            bufs[4 * m + c], sem))
                for cp in copies:
                    cp.wait()

                def vert_body(j, jcarry):
                    zero16 = lane * 0
                    cmask = lane < 3
                    cvals = plsc.load_gather(
                        coord_v, [j * 3 + jnp.minimum(lane, 2)], mask=cmask)
                    plsc.store_scatter(out_v, [j * _OUT_D + lane], cvals,
                                       mask=cmask)
                    for m in range(4):
                        ws = [plsc.load_gather(
                                  w_v, [zero16 + ((4 * m + c) * 16 + j)])
                              for c in range(4)]
                        ch = _CH[m]
                        off = _COL[m]
                        b0, b1, b2, b3 = (bufs[4 * m + c] for c in range(4))

                        def ch_body(kk, kcarry, ws=ws, b0=b0, b1=b1, b2=b2,
                                    b3=b3, off=off):
                            sl = pl.ds(kk * 16, 16)
                            acc = (ws[0] * b0[j, sl] + ws[1] * b1[j, sl]
                                   + ws[2] * b2[j, sl] + ws[3] * b3[j, sl])
                            out_v[pl.ds(j * _OUT_D + off + kk * 16, 16)] = acc
                            return kcarry

                        lax.fori_loop(0, ch // 16, ch_body, 0)
                    return jcarry

                lax.fori_loop(0, _B, vert_body, 0)
                pltpu.sync_copy(out_v,
                                out_hbm.at[pl.ds(vbase * _OUT_D, _B * _OUT_D)])

            return carry

        lax.fori_loop(0, trips, trip_body, 0)

    return sc_kernel


def kernel(inputs, img_feat0, img_feat1, img_feat2, img_feat3):
    n = inputs.shape[0]
    t0 = img_feat0.reshape(_SS[0] * _SS[0], _CH[0])
    t1 = img_feat1.reshape(_SS[1] * _SS[1], _CH[1])
    t2 = img_feat2.reshape(_SS[2] * _SS[2], _CH[2])
    t3 = img_feat3.reshape(_SS[3] * _SS[3], _CH[3])
    flat = _make_sc_kernel(n)(inputs.reshape(n * 3), t0, t1, t2, t3)
    return flat.reshape(n, _OUT_D)


# speedup vs baseline: 3.2756x; 1.0027x over previous
import functools

import jax
import jax.numpy as jnp
from jax import lax
from jax.experimental import pallas as pl
from jax.experimental.pallas import tpu as pltpu
from jax.experimental.pallas import tpu_sc as plsc

_B = 16
_CH = (64, 128, 256, 512)
_SS = (56, 28, 14, 7)
_COL = (3, 67, 195, 451)
_OUT_D = 963


@functools.lru_cache(maxsize=None)
def _make_sc_kernel(n):
    assert n % (2 * _B) == 0
    info = plsc.get_sparse_core_info()
    nc, ns = info.num_cores, info.num_subcores
    nw = nc * ns
    nblk = n // _B
    trips = (nblk + nw - 1) // nw
    if trips % 2:
        trips += 1

    mesh = plsc.VectorSubcoreMesh(core_axis_name="c", subcore_axis_name="s")

    scratch = [
        pltpu.VMEM((_B * 3,), jnp.float32),
        pltpu.VMEM((_B * 3,), jnp.float32),
        pltpu.VMEM((4, 4 * _B), jnp.int32),
        pltpu.VMEM((4, 4 * _B), jnp.int32),
        pltpu.VMEM((16 * _B,), jnp.float32),
        pltpu.VMEM((16 * _B,), jnp.float32),
        pltpu.VMEM((_B * _OUT_D,), jnp.float32),
        pltpu.VMEM((_B * _OUT_D,), jnp.float32),
    ]
    for c in _CH:
        scratch.append(pltpu.VMEM((4 * _B, c), jnp.float32))
    scratch += [pltpu.SemaphoreType.DMA] * 8

    @functools.partial(
        pl.kernel,
        mesh=mesh,
        out_type=jax.ShapeDtypeStruct((n * _OUT_D,), jnp.float32),
        scratch_types=scratch,
        compiler_params=pltpu.CompilerParams(
            needs_layout_passes=False, use_tc_tiling_on_sc=False),
    )
    def sc_kernel(coord_hbm, t0, t1, t2, t3, out_hbm,
                  cv0, cv1, ix0, ix1, wv0, wv1, ov0, ov1,
                  b0, b1, b2, b3,
                  sc0, sc1, sg0, sg1, sg2, sg3, so0, so1):
        coord_v = (cv0, cv1)
        idx_v = (ix0, ix1)
        w_v = (wv0, wv1)
        out_v = (ov0, ov1)
        bufs = (b0, b1, b2, b3)
        sem_c = (sc0, sc1)
        sem_g = (sg0, sg1, sg2, sg3)
        sem_o = (so0, so1)
        tables = (t0, t1, t2, t3)
        wid = lax.axis_index("s") * nc + lax.axis_index("c")
        lane = lax.iota(jnp.int32, 16)

        def bid_of(t):
            return t * nw + wid

        def fire_coord(t, q):
            pltpu.async_copy(
                coord_hbm.at[pl.ds(bid_of(t) * _B * 3, _B * 3)],
                coord_v[q], sem_c[q])

        def compute_idx(q):
            lane3 = lane * 3
            xc = plsc.load_gather(coord_v[q], [lane3])
            yc = plsc.load_gather(coord_v[q], [lane3 + 1])
            zc = plsc.load_gather(coord_v[q], [lane3 + 2])
            nz = -zc
            h = 250.0 * (-yc / nz) + 112.0
            w = 250.0 * (xc / nz) + 112.0
            h = jnp.minimum(jnp.maximum(h, 0.0), 223.0)
            w = jnp.minimum(jnp.maximum(w, 0.0), 223.0)
            for m in range(4):
                s = _SS[m]
                x = h * (s / 224.0)
                y = w * (s / 224.0)
                xi1 = x.astype(jnp.int32)
                x1f = xi1.astype(jnp.float32)
                xi2 = jnp.minimum(xi1 + (x > x1f).astype(jnp.int32), s - 1)
                x2f = xi2.astype(jnp.float32)
                yi1 = y.astype(jnp.int32)
                y1f = yi1.astype(jnp.float32)
                yi2 = jnp.minimum(yi1 + (y > y1f).astype(jnp.int32), s - 1)
                y2f = yi2.astype(jnp.float32)
                idx_v[q][m, pl.ds(0, 16)] = xi1 * s + yi1
                idx_v[q][m, pl.ds(16, 16)] = xi2 * s + yi1
                idx_v[q][m, pl.ds(32, 16)] = xi1 * s + yi2
                idx_v[q][m, pl.ds(48, 16)] = xi2 * s + yi2
                w_v[q][pl.ds((4 * m + 0) * 16, 16)] = (x2f - x) * (y2f - y)
                w_v[q][pl.ds((4 * m + 1) * 16, 16)] = (x - x1f) * (y2f - y)
                w_v[q][pl.ds((4 * m + 2) * 16, 16)] = (x2f - x) * (y - y1f)
                w_v[q][pl.ds((4 * m + 3) * 16, 16)] = (x - x1f) * (y - y1f)

        def fire_gather(m, q):
            pltpu.async_copy(tables[m].at[idx_v[q].at[m]], bufs[m], sem_g[m])

        def wait_gather(m, q):
            pltpu.make_async_copy(
                tables[m].at[idx_v[q].at[m]], bufs[m], sem_g[m]).wait()

        def fire_out(t, p):
            pltpu.async_copy(
                out_v[p],
                out_hbm.at[pl.ds(bid_of(t) * _B * _OUT_D, _B * _OUT_D)],
                sem_o[p])

        def wait_out(p):
            pltpu.make_async_copy(
                out_v[p],
                out_hbm.at[pl.ds(0, _B * _OUT_D)], sem_o[p]).wait()

        def accum_map(m, p):
            ch = _CH[m]
            off = _COL[m]
            bm = bufs[m]
            wp = w_v[p]
            ovp = out_v[p]
            zero16 = lane * 0

            def vert_body(j, jcarry):
                ws = [plsc.load_gather(wp, [zero16 + ((4 * m + c) * 16 + j)])
                      for c in range(4)]

                def ch_body(kk, kcarry):
                    sl = pl.ds(kk * 16, 16)
                    acc = (ws[0] * bm[0 * 16 + j, sl]
                           + ws[1] * bm[1 * 16 + j, sl]
                           + ws[2] * bm[2 * 16 + j, sl]
                           + ws[3] * bm[3 * 16 + j, sl])
                    ovp[pl.ds(j * _OUT_D + off + kk * 16, 16)] = acc
                    return kcarry

                lax.fori_loop(0, ch // 16, ch_body, 0)
                return jcarry

            lax.fori_loop(0, _B, vert_body, 0)

        def coord_pass(p):
            cmask = lane < 3

            def vert_body(j, jcarry):
                cvals = plsc.load_gather(
                    coord_v[p], [j * 3 + jnp.minimum(lane, 2)], mask=cmask)
                plsc.store_scatter(out_v[p], [j * _OUT_D + lane], cvals,
                                   mask=cmask)
                return jcarry

            lax.fori_loop(0, _B, vert_body, 0)

        def phase(t, i, p):
            q = 1 - p
            tn = t + 1

            @pl.when(bid_of(t) < nblk)
            def _():
                @pl.when(bid_of(tn) < nblk)
                def _():
                    fire_coord(tn, q)

                wait_gather(0, p)

                @pl.when(i >= 1)
                def _():
                    wait_out(p)

                coord_pass(p)
                accum_map(0, p)

                @pl.when(bid_of(tn) < nblk)
                def _():
                    pltpu.make_async_copy(
                        coord_hbm.at[pl.ds(0, _B * 3)],
                        coord_v[q], sem_c[q]).wait()
                    compute_idx(q)
                    fire_gather(0, q)

                for m in range(1, 4):
                    wait_gather(m, p)
                    accum_map(m, p)

                    @pl.when(bid_of(tn) < nblk)
                    def _(m=m):
                        fire_gather(m, q)

                fire_out(t, p)

        pltpu.sync_copy(coord_hbm.at[pl.ds(bid_of(0) * _B * 3, _B * 3)],
                        coord_v[0])
        compute_idx(0)
        for m in range(4):
            fire_gather(m, 0)

        def trip_body(i, carry):
            phase(2 * i, i, 0)
            phase(2 * i + 1, i, 1)
            return carry

        lax.fori_loop(0, trips // 2, trip_body, 0)
        wait_out(0)
        wait_out(1)

    return sc_kernel


def kernel(inputs, img_feat0, img_feat1, img_feat2, img_feat3):
    n = inputs.shape[0]
    t0 = img_feat0.reshape(_SS[0] * _SS[0], _CH[0])
    t1 = img_feat1.reshape(_SS[1] * _SS[1], _CH[1])
    t2 = img_feat2.reshape(_SS[2] * _SS[2], _CH[2])
    t3 = img_feat3.reshape(_SS[3] * _SS[3], _CH[3])
    flat = _make_sc_kernel(n)(inputs.reshape(n * 3), t0, t1, t2, t3)
    return flat.reshape(n, _OUT_D)


# speedup vs baseline: 3.2889x; 1.0041x over previous
import functools

import jax
import jax.numpy as jnp
from jax import lax
from jax.experimental import pallas as pl
from jax.experimental.pallas import tpu as pltpu
from jax.experimental.pallas import tpu_sc as plsc

_B = 16
_CH = (64, 128, 256, 512)
_SS = (56, 28, 14, 7)
_COL = (3, 67, 195, 451)
_OUT_D = 963


@functools.lru_cache(maxsize=None)
def _make_sc_kernel(n):
    assert n % (2 * _B) == 0
    info = plsc.get_sparse_core_info()
    nc, ns = info.num_cores, info.num_subcores
    nw = nc * ns
    nblk = n // _B
    trips = (nblk + nw - 1) // nw
    if trips % 2:
        trips += 1

    mesh = plsc.VectorSubcoreMesh(core_axis_name="c", subcore_axis_name="s")

    scratch = [
        pltpu.VMEM((_B * 3,), jnp.float32),
        pltpu.VMEM((_B * 3,), jnp.float32),
        pltpu.VMEM((4, 4 * _B), jnp.int32),
        pltpu.VMEM((4, 4 * _B), jnp.int32),
        pltpu.VMEM((16 * _B,), jnp.float32),
        pltpu.VMEM((16 * _B,), jnp.float32),
        pltpu.VMEM((_B * _OUT_D,), jnp.float32),
        pltpu.VMEM((_B * _OUT_D,), jnp.float32),
    ]
    for c in _CH:
        scratch.append(pltpu.VMEM((4 * _B, c), jnp.float32))
    scratch += [pltpu.SemaphoreType.DMA] * 8

    @functools.partial(
        pl.kernel,
        mesh=mesh,
        out_type=jax.ShapeDtypeStruct((n * _OUT_D,), jnp.float32),
        scratch_types=scratch,
        compiler_params=pltpu.CompilerParams(
            needs_layout_passes=False, use_tc_tiling_on_sc=False),
    )
    def sc_kernel(coord_hbm, t0, t1, t2, t3, out_hbm,
                  cv0, cv1, ix0, ix1, wv0, wv1, ov0, ov1,
                  b0, b1, b2, b3,
                  sc0, sc1, sg0, sg1, sg2, sg3, so0, so1):
        coord_v = (cv0, cv1)
        idx_v = (ix0, ix1)
        w_v = (wv0, wv1)
        out_v = (ov0, ov1)
        bufs = (b0, b1, b2, b3)
        sem_c = (sc0, sc1)
        sem_g = (sg0, sg1, sg2, sg3)
        sem_o = (so0, so1)
        tables = (t0, t1, t2, t3)
        wid = lax.axis_index("s") * nc + lax.axis_index("c")
        lane = lax.iota(jnp.int32, 16)

        def bid_of(t):
            return t * nw + wid

        def fire_coord(t, q):
            pltpu.async_copy(
                coord_hbm.at[pl.ds(bid_of(t) * _B * 3, _B * 3)],
                coord_v[q], sem_c[q])

        def compute_idx(q):
            lane3 = lane * 3
            xc = plsc.load_gather(coord_v[q], [lane3])
            yc = plsc.load_gather(coord_v[q], [lane3 + 1])
            zc = plsc.load_gather(coord_v[q], [lane3 + 2])
            nz = -zc
            h = 250.0 * (-yc / nz) + 112.0
            w = 250.0 * (xc / nz) + 112.0
            h = jnp.minimum(jnp.maximum(h, 0.0), 223.0)
            w = jnp.minimum(jnp.maximum(w, 0.0), 223.0)
            for m in range(4):
                s = _SS[m]
                x = h * (s / 224.0)
                y = w * (s / 224.0)
                xi1 = x.astype(jnp.int32)
                x1f = xi1.astype(jnp.float32)
                xi2 = jnp.minimum(xi1 + (x > x1f).astype(jnp.int32), s - 1)
                x2f = xi2.astype(jnp.float32)
                yi1 = y.astype(jnp.int32)
                y1f = yi1.astype(jnp.float32)
                yi2 = jnp.minimum(yi1 + (y > y1f).astype(jnp.int32), s - 1)
                y2f = yi2.astype(jnp.float32)
                idx_v[q][m, pl.ds(0, 16)] = xi1 * s + yi1
                idx_v[q][m, pl.ds(16, 16)] = xi2 * s + yi1
                idx_v[q][m, pl.ds(32, 16)] = xi1 * s + yi2
                idx_v[q][m, pl.ds(48, 16)] = xi2 * s + yi2
                w_v[q][pl.ds((4 * m + 0) * 16, 16)] = (x2f - x) * (y2f - y)
                w_v[q][pl.ds((4 * m + 1) * 16, 16)] = (x - x1f) * (y2f - y)
                w_v[q][pl.ds((4 * m + 2) * 16, 16)] = (x2f - x) * (y - y1f)
                w_v[q][pl.ds((4 * m + 3) * 16, 16)] = (x - x1f) * (y - y1f)

        def fire_gather(m, q):
            pltpu.async_copy(tables[m].at[idx_v[q].at[m]], bufs[m], sem_g[m])

        def wait_gather(m, q):
            pltpu.make_async_copy(
                tables[m].at[idx_v[q].at[m]], bufs[m], sem_g[m]).wait()

        def fire_out(t, p):
            pltpu.async_copy(
                out_v[p],
                out_hbm.at[pl.ds(bid_of(t) * _B * _OUT_D, _B * _OUT_D)],
                sem_o[p])

        def wait_out(p):
            pltpu.make_async_copy(
                out_v[p],
                out_hbm.at[pl.ds(0, _B * _OUT_D)], sem_o[p]).wait()

        def accum_map(m, p):
            ch = _CH[m]
            off = _COL[m]
            bm = bufs[m]
            wp = w_v[p]
            ovp = out_v[p]
            zero16 = lane * 0

            @plsc.parallel_loop(0, _B, unroll=2)
            def vert_body(j):
                ws = [plsc.load_gather(wp, [zero16 + ((4 * m + c) * 16 + j)])
                      for c in range(4)]
                for kk in range(ch // 16):
                    sl = pl.ds(kk * 16, 16)
                    acc = (ws[0] * bm[0 * 16 + j, sl]
                           + ws[1] * bm[1 * 16 + j, sl]
                           + ws[2] * bm[2 * 16 + j, sl]
                           + ws[3] * bm[3 * 16 + j, sl])
                    ovp[pl.ds(j * _OUT_D + off + kk * 16, 16)] = acc

        def coord_pass(p):
            cmask = lane < 3

            @plsc.parallel_loop(0, _B, unroll=2)
            def vert_body(j):
                cvals = plsc.load_gather(
                    coord_v[p], [j * 3 + jnp.minimum(lane, 2)], mask=cmask)
                plsc.store_scatter(out_v[p], [j * _OUT_D + lane], cvals,
                                   mask=cmask)

        def phase(t, i, p):
            q = 1 - p
            tn = t + 1

            @pl.when(bid_of(t) < nblk)
            def _():
                @pl.when(bid_of(tn) < nblk)
                def _():
                    fire_coord(tn, q)

                wait_gather(0, p)

                @pl.when(i >= 1)
                def _():
                    wait_out(p)

                coord_pass(p)
                accum_map(0, p)

                @pl.when(bid_of(tn) < nblk)
                def _():
                    pltpu.make_async_copy(
                        coord_hbm.at[pl.ds(0, _B * 3)],
                        coord_v[q], sem_c[q]).wait()
                    compute_idx(q)
                    fire_gather(0, q)

                for m in range(1, 4):
                    wait_gather(m, p)
                    accum_map(m, p)

                    @pl.when(bid_of(tn) < nblk)
                    def _(m=m):
                        fire_gather(m, q)

                fire_out(t, p)

        pltpu.sync_copy(coord_hbm.at[pl.ds(bid_of(0) * _B * 3, _B * 3)],
                        coord_v[0])
        compute_idx(0)
        for m in range(4):
            fire_gather(m, 0)

        def trip_body(i, carry):
            phase(2 * i, i, 0)
            phase(2 * i + 1, i, 1)
            return carry

        lax.fori_loop(0, trips // 2, trip_body, 0)
        wait_out(0)
        wait_out(1)

    return sc_kernel


def kernel(inputs, img_feat0, img_feat1, img_feat2, img_feat3):
    n = inputs.shape[0]
    t0 = img_feat0.reshape(_SS[0] * _SS[0], _CH[0])
    t1 = img_feat1.reshape(_SS[1] * _SS[1], _CH[1])
    t2 = img_feat2.reshape(_SS[2] * _SS[2], _CH[2])
    t3 = img_feat3.reshape(_SS[3] * _SS[3], _CH[3])
    flat = _make_sc_kernel(n)(inputs.reshape(n * 3), t0, t1, t2, t3)
    return flat.reshape(n, _OUT_D)


# speedup vs baseline: 4.7482x; 1.4437x over previous
import functools

import jax
import jax.numpy as jnp
from jax import lax
from jax.experimental import pallas as pl
from jax.experimental.pallas import tpu as pltpu
from jax.experimental.pallas import tpu_sc as plsc

_B = 16
_CH = (64, 128, 256, 512)
_SS = (56, 28, 14, 7)
_COL = (3, 67, 195, 451)
_OUT_D = 963


@functools.lru_cache(maxsize=None)
def _make_sc_kernel(n):
    assert n % (2 * _B) == 0
    info = plsc.get_sparse_core_info()
    nc, ns = info.num_cores, info.num_subcores
    nw = nc * ns
    nblk = n // _B
    trips = (nblk + nw - 1) // nw
    if trips % 2:
        trips += 1

    mesh = plsc.VectorSubcoreMesh(core_axis_name="c", subcore_axis_name="s")

    scratch = [
        pltpu.VMEM((_B * 3,), jnp.float32),
        pltpu.VMEM((_B * 3,), jnp.float32),
        pltpu.VMEM((4, 2 * _B), jnp.int32),
        pltpu.VMEM((4, 2 * _B), jnp.int32),
        pltpu.VMEM((16 * _B,), jnp.float32),
        pltpu.VMEM((16 * _B,), jnp.float32),
        pltpu.VMEM((_B * _OUT_D,), jnp.float32),
        pltpu.VMEM((_B * _OUT_D,), jnp.float32),
    ]
    for c in _CH:
        scratch.append(pltpu.VMEM((2 * _B, 2 * c), jnp.float32))
    scratch += [pltpu.SemaphoreType.DMA] * 8

    @functools.partial(
        pl.kernel,
        mesh=mesh,
        out_type=jax.ShapeDtypeStruct((n * _OUT_D,), jnp.float32),
        scratch_types=scratch,
        compiler_params=pltpu.CompilerParams(
            needs_layout_passes=False, use_tc_tiling_on_sc=False),
    )
    def sc_kernel(coord_hbm, t0, t1, t2, t3, out_hbm,
                  cv0, cv1, ix0, ix1, wv0, wv1, ov0, ov1,
                  b0, b1, b2, b3,
                  sc0, sc1, sg0, sg1, sg2, sg3, so0, so1):
        coord_v = (cv0, cv1)
        idx_v = (ix0, ix1)
        w_v = (wv0, wv1)
        out_v = (ov0, ov1)
        bufs = (b0, b1, b2, b3)
        sem_c = (sc0, sc1)
        sem_g = (sg0, sg1, sg2, sg3)
        sem_o = (so0, so1)
        tables = (t0, t1, t2, t3)
        wid = lax.axis_index("s") * nc + lax.axis_index("c")
        lane = lax.iota(jnp.int32, 16)

        def bid_of(t):
            return t * nw + wid

        def fire_coord(t, q):
            pltpu.async_copy(
                coord_hbm.at[pl.ds(bid_of(t) * _B * 3, _B * 3)],
                coord_v[q], sem_c[q])

        def compute_idx(q):
            lane3 = lane * 3
            xc = plsc.load_gather(coord_v[q], [lane3])
            yc = plsc.load_gather(coord_v[q], [lane3 + 1])
            zc = plsc.load_gather(coord_v[q], [lane3 + 2])
            nz = -zc
            h = 250.0 * (-yc / nz) + 112.0
            w = 250.0 * (xc / nz) + 112.0
            h = jnp.minimum(jnp.maximum(h, 0.0), 223.0)
            w = jnp.minimum(jnp.maximum(w, 0.0), 223.0)
            for m in range(4):
                s = _SS[m]
                x = h * (s / 224.0)
                y = w * (s / 224.0)
                xi1 = x.astype(jnp.int32)
                x1f = xi1.astype(jnp.float32)
                xi2 = jnp.minimum(xi1 + (x > x1f).astype(jnp.int32), s - 1)
                x2f = xi2.astype(jnp.float32)
                yi1 = y.astype(jnp.int32)
                y1f = yi1.astype(jnp.float32)
                yi2 = jnp.minimum(yi1 + (y > y1f).astype(jnp.int32), s - 1)
                y2f = yi2.astype(jnp.float32)
                idx_v[q][m, pl.ds(0, 16)] = xi1 * s + yi1
                idx_v[q][m, pl.ds(16, 16)] = xi2 * s + yi1
                w11 = (x2f - x) * (y2f - y)
                w21 = (x - x1f) * (y2f - y)
                w12 = (x2f - x) * (y - y1f)
                w22 = (x - x1f) * (y - y1f)
                deg = yi2 == yi1
                zf = w11 * 0.0
                w_v[q][pl.ds((4 * m + 0) * 16, 16)] = (
                    w11 + jnp.where(deg, w12, zf))
                w_v[q][pl.ds((4 * m + 1) * 16, 16)] = (
                    w21 + jnp.where(deg, w22, zf))
                w_v[q][pl.ds((4 * m + 2) * 16, 16)] = jnp.where(deg, zf, w12)
                w_v[q][pl.ds((4 * m + 3) * 16, 16)] = jnp.where(deg, zf, w22)

        def fire_gather(m, q):
            pltpu.async_copy(tables[m].at[idx_v[q].at[m]], bufs[m], sem_g[m])

        def wait_gather(m, q):
            pltpu.make_async_copy(
                tables[m].at[idx_v[q].at[m]], bufs[m], sem_g[m]).wait()

        def fire_out(t, p):
            pltpu.async_copy(
                out_v[p],
                out_hbm.at[pl.ds(bid_of(t) * _B * _OUT_D, _B * _OUT_D)],
                sem_o[p])

        def wait_out(p):
            pltpu.make_async_copy(
                out_v[p],
                out_hbm.at[pl.ds(0, _B * _OUT_D)], sem_o[p]).wait()

        def accum_map(m, p):
            ch = _CH[m]
            off = _COL[m]
            bm = bufs[m]
            wp = w_v[p]
            ovp = out_v[p]
            zero16 = lane * 0

            @plsc.parallel_loop(0, _B, unroll=2)
            def vert_body(j):
                ws = [plsc.load_gather(wp, [zero16 + ((4 * m + c) * 16 + j)])
                      for c in range(4)]
                for kk in range(ch // 16):
                    sl = pl.ds(kk * 16, 16)
                    sh = pl.ds(ch + kk * 16, 16)
                    acc = (ws[0] * bm[0 * 16 + j, sl]
                           + ws[1] * bm[1 * 16 + j, sl]
                           + ws[2] * bm[0 * 16 + j, sh]
                           + ws[3] * bm[1 * 16 + j, sh])
                    ovp[pl.ds(j * _OUT_D + off + kk * 16, 16)] = acc

        def coord_pass(p):
            cmask = lane < 3

            @plsc.parallel_loop(0, _B, unroll=2)
            def vert_body(j):
                cvals = plsc.load_gather(
                    coord_v[p], [j * 3 + jnp.minimum(lane, 2)], mask=cmask)
                plsc.store_scatter(out_v[p], [j * _OUT_D + lane], cvals,
                                   mask=cmask)

        def phase(t, i, p):
            q = 1 - p
            tn = t + 1

            @pl.when(bid_of(t) < nblk)
            def _():
                @pl.when(bid_of(tn) < nblk)
                def _():
                    fire_coord(tn, q)

                wait_gather(0, p)

                @pl.when(i >= 1)
                def _():
                    wait_out(p)

                coord_pass(p)
                accum_map(0, p)

                @pl.when(bid_of(tn) < nblk)
                def _():
                    pltpu.make_async_copy(
                        coord_hbm.at[pl.ds(0, _B * 3)],
                        coord_v[q], sem_c[q]).wait()
                    compute_idx(q)
                    fire_gather(0, q)

                for m in range(1, 4):
                    wait_gather(m, p)
                    accum_map(m, p)

                    @pl.when(bid_of(tn) < nblk)
                    def _(m=m):
                        fire_gather(m, q)

                fire_out(t, p)

        pltpu.sync_copy(coord_hbm.at[pl.ds(bid_of(0) * _B * 3, _B * 3)],
                        coord_v[0])
        compute_idx(0)
        for m in range(4):
            fire_gather(m, 0)

        def trip_body(i, carry):
            phase(2 * i, i, 0)
            phase(2 * i + 1, i, 1)
            return carry

        lax.fori_loop(0, trips // 2, trip_body, 0)
        wait_out(0)
        wait_out(1)

    return sc_kernel


def _pair_table(feat, s, c):
    t = feat.reshape(s * s, c)
    return jnp.concatenate([t, jnp.roll(t, -1, axis=0)], axis=1)


def kernel(inputs, img_feat0, img_feat1, img_feat2, img_feat3):
    n = inputs.shape[0]
    t0 = _pair_table(img_feat0, _SS[0], _CH[0])
    t1 = _pair_table(img_feat1, _SS[1], _CH[1])
    t2 = _pair_table(img_feat2, _SS[2], _CH[2])
    t3 = _pair_table(img_feat3, _SS[3], _CH[3])
    flat = _make_sc_kernel(n)(inputs.reshape(n * 3), t0, t1, t2, t3)
    return flat.reshape(n, _OUT_D)


# speedup vs baseline: 5.2454x; 1.1047x over previous
import functools

import jax
import jax.numpy as jnp
from jax import lax
from jax.experimental import pallas as pl
from jax.experimental.pallas import tpu as pltpu
from jax.experimental.pallas import tpu_sc as plsc

_B = 16
_CH = (64, 128, 256, 512)
_SS = (56, 28, 14, 7)
_COL = (3, 67, 195, 451)
_OUT_D = 963


@functools.lru_cache(maxsize=None)
def _make_sc_kernel(n):
    assert n % (2 * _B) == 0
    info = plsc.get_sparse_core_info()
    nc, ns = info.num_cores, info.num_subcores
    nw = nc * ns
    nblk = n // _B
    trips = (nblk + nw - 1) // nw
    if trips % 2:
        trips += 1

    mesh = plsc.VectorSubcoreMesh(core_axis_name="c", subcore_axis_name="s")

    scratch = [
        pltpu.VMEM((_B * 3,), jnp.float32),
        pltpu.VMEM((_B * 3,), jnp.float32),
        pltpu.VMEM((4, _B), jnp.int32),
        pltpu.VMEM((4, _B), jnp.int32),
        pltpu.VMEM((16 * _B,), jnp.float32),
        pltpu.VMEM((16 * _B,), jnp.float32),
        pltpu.VMEM((_B * _OUT_D,), jnp.float32),
        pltpu.VMEM((_B * _OUT_D,), jnp.float32),
    ]
    for c in _CH:
        scratch.append(pltpu.VMEM((_B, 4 * c), jnp.float32))
    scratch += [pltpu.SemaphoreType.DMA] * 8

    @functools.partial(
        pl.kernel,
        mesh=mesh,
        out_type=jax.ShapeDtypeStruct((n * _OUT_D,), jnp.float32),
        scratch_types=scratch,
        compiler_params=pltpu.CompilerParams(
            needs_layout_passes=False, use_tc_tiling_on_sc=False),
    )
    def sc_kernel(coord_hbm, t0, t1, t2, t3, out_hbm,
                  cv0, cv1, ix0, ix1, wv0, wv1, ov0, ov1,
                  b0, b1, b2, b3,
                  sc0, sc1, sg0, sg1, sg2, sg3, so0, so1):
        coord_v = (cv0, cv1)
        idx_v = (ix0, ix1)
        w_v = (wv0, wv1)
        out_v = (ov0, ov1)
        bufs = (b0, b1, b2, b3)
        sem_c = (sc0, sc1)
        sem_g = (sg0, sg1, sg2, sg3)
        sem_o = (so0, so1)
        tables = (t0, t1, t2, t3)
        wid = lax.axis_index("s") * nc + lax.axis_index("c")
        lane = lax.iota(jnp.int32, 16)

        def bid_of(t):
            return t * nw + wid

        def fire_coord(t, q):
            pltpu.async_copy(
                coord_hbm.at[pl.ds(bid_of(t) * _B * 3, _B * 3)],
                coord_v[q], sem_c[q])

        def compute_idx(q):
            lane3 = lane * 3
            xc = plsc.load_gather(coord_v[q], [lane3])
            yc = plsc.load_gather(coord_v[q], [lane3 + 1])
            zc = plsc.load_gather(coord_v[q], [lane3 + 2])
            nz = -zc
            h = 250.0 * (-yc / nz) + 112.0
            w = 250.0 * (xc / nz) + 112.0
            h = jnp.minimum(jnp.maximum(h, 0.0), 223.0)
            w = jnp.minimum(jnp.maximum(w, 0.0), 223.0)
            for m in range(4):
                s = _SS[m]
                x = h * (s / 224.0)
                y = w * (s / 224.0)
                xi1 = x.astype(jnp.int32)
                x1f = xi1.astype(jnp.float32)
                xi2 = jnp.minimum(xi1 + (x > x1f).astype(jnp.int32), s - 1)
                x2f = xi2.astype(jnp.float32)
                yi1 = y.astype(jnp.int32)
                y1f = yi1.astype(jnp.float32)
                yi2 = jnp.minimum(yi1 + (y > y1f).astype(jnp.int32), s - 1)
                y2f = yi2.astype(jnp.float32)
                idx_v[q][m, :] = xi1 * s + yi1
                w11 = (x2f - x) * (y2f - y)
                w21 = (x - x1f) * (y2f - y)
                w12 = (x2f - x) * (y - y1f)
                w22 = (x - x1f) * (y - y1f)
                ydeg = yi2 == yi1
                xdeg = xi2 == xi1
                zf = w11 * 0.0
                w_v[q][pl.ds((4 * m + 0) * 16, 16)] = (
                    w11 + jnp.where(ydeg, w12, zf)
                    + jnp.where(xdeg, w21 + jnp.where(ydeg, w22, zf), zf))
                w_v[q][pl.ds((4 * m + 1) * 16, 16)] = jnp.where(
                    xdeg, zf, w21 + jnp.where(ydeg, w22, zf))
                w_v[q][pl.ds((4 * m + 2) * 16, 16)] = jnp.where(
                    ydeg, zf, w12 + jnp.where(xdeg, w22, zf))
                w_v[q][pl.ds((4 * m + 3) * 16, 16)] = jnp.where(
                    ydeg | xdeg, zf, w22)

        def fire_gather(m, q):
            pltpu.async_copy(tables[m].at[idx_v[q].at[m]], bufs[m], sem_g[m])

        def wait_gather(m, q):
            pltpu.make_async_copy(
                tables[m].at[idx_v[q].at[m]], bufs[m], sem_g[m]).wait()

        def fire_out(t, p):
            pltpu.async_copy(
                out_v[p],
                out_hbm.at[pl.ds(bid_of(t) * _B * _OUT_D, _B * _OUT_D)],
                sem_o[p])

        def wait_out(p):
            pltpu.make_async_copy(
                out_v[p],
                out_hbm.at[pl.ds(0, _B * _OUT_D)], sem_o[p]).wait()

        def accum_map(m, p):
            ch = _CH[m]
            off = _COL[m]
            bm = bufs[m]
            wp = w_v[p]
            ovp = out_v[p]
            zero16 = lane * 0

            @plsc.parallel_loop(0, _B, unroll=2)
            def vert_body(j):
                ws = [plsc.load_gather(wp, [zero16 + ((4 * m + c) * 16 + j)])
                      for c in range(4)]
                for kk in range(ch // 16):
                    acc = (ws[0] * bm[j, pl.ds(kk * 16, 16)]
                           + ws[1] * bm[j, pl.ds(2 * ch + kk * 16, 16)]
                           + ws[2] * bm[j, pl.ds(ch + kk * 16, 16)]
                           + ws[3] * bm[j, pl.ds(3 * ch + kk * 16, 16)])
                    ovp[pl.ds(j * _OUT_D + off + kk * 16, 16)] = acc

        def coord_pass(p):
            cmask = lane < 3

            @plsc.parallel_loop(0, _B, unroll=2)
            def vert_body(j):
                cvals = plsc.load_gather(
                    coord_v[p], [j * 3 + jnp.minimum(lane, 2)], mask=cmask)
                plsc.store_scatter(out_v[p], [j * _OUT_D + lane], cvals,
                                   mask=cmask)

        def phase(t, i, p):
            q = 1 - p
            tn = t + 1

            @pl.when(bid_of(t) < nblk)
            def _():
                @pl.when(bid_of(tn) < nblk)
                def _():
                    fire_coord(tn, q)

                wait_gather(0, p)

                @pl.when(i >= 1)
                def _():
                    wait_out(p)

                coord_pass(p)
                accum_map(0, p)

                @pl.when(bid_of(tn) < nblk)
                def _():
                    pltpu.make_async_copy(
                        coord_hbm.at[pl.ds(0, _B * 3)],
                        coord_v[q], sem_c[q]).wait()
                    compute_idx(q)
                    fire_gather(0, q)

                for m in range(1, 4):
                    wait_gather(m, p)
                    accum_map(m, p)

                    @pl.when(bid_of(tn) < nblk)
                    def _(m=m):
                        fire_gather(m, q)

                fire_out(t, p)

        pltpu.sync_copy(coord_hbm.at[pl.ds(bid_of(0) * _B * 3, _B * 3)],
                        coord_v[0])
        compute_idx(0)
        for m in range(4):
            fire_gather(m, 0)

        def trip_body(i, carry):
            phase(2 * i, i, 0)
            phase(2 * i + 1, i, 1)
            return carry

        lax.fori_loop(0, trips // 2, trip_body, 0)
        wait_out(0)
        wait_out(1)

    return sc_kernel


def _quad_table(feat, s, c):
    t = feat.reshape(s * s, c)
    return jnp.concatenate(
        [t, jnp.roll(t, -1, axis=0),
         jnp.roll(t, -s, axis=0), jnp.roll(t, -s - 1, axis=0)], axis=1)


def kernel(inputs, img_feat0, img_feat1, img_feat2, img_feat3):
    n = inputs.shape[0]
    t0 = _quad_table(img_feat0, _SS[0], _CH[0])
    t1 = _quad_table(img_feat1, _SS[1], _CH[1])
    t2 = _quad_table(img_feat2, _SS[2], _CH[2])
    t3 = _quad_table(img_feat3, _SS[3], _CH[3])
    flat = _make_sc_kernel(n)(inputs.reshape(n * 3), t0, t1, t2, t3)
    return flat.reshape(n, _OUT_D)
